# Initial kernel scaffold; baseline (speedup 1.0000x reference)
import jax
import jax.numpy as jnp
from jax.experimental import pallas as pl


def kernel(xyz, points):
    raise NotImplementedError("write your pallas kernel here")


# speedup vs baseline: 1.1076x; 1.1076x over previous
import functools

import jax
import jax.numpy as jnp
from jax import lax
from jax.experimental import pallas as pl
from jax.experimental.pallas import tpu as pltpu

NPOINT = 1024
RADIUS = 0.2
NSAMPLE = 64
B = 8
N = 4096
SEED_ROWS = 64
NUM_SEED_TILES = N // SEED_ROWS


def _first_argmax(v, iota, axis):
    m = jnp.max(v, axis=axis, keepdims=True)
    return m, jnp.min(jnp.where(v == m, iota, jnp.int32(N * N)), axis=axis,
                      keepdims=True)


def _fps_kernel(lhs_seed_ref, nseed_ref, xyzT24_ref, x_ref, y_ref, z_ref,
                n8_ref, selx_ref, sely_ref, selz_ref):
    n8 = n8_ref[...]
    iota_n = lax.broadcasted_iota(jnp.int32, (B, N), 1)

    def dist_row(i):
        onehot = iota_n == i
        xs = jnp.sum(jnp.where(onehot, x_ref[...], 0.0), axis=1, keepdims=True)
        ys = jnp.sum(jnp.where(onehot, y_ref[...], 0.0), axis=1, keepdims=True)
        zs = jnp.sum(jnp.where(onehot, z_ref[...], 0.0), axis=1, keepdims=True)
        ns = jnp.sum(jnp.where(onehot, n8, 0.0), axis=1, keepdims=True)
        lane = lax.broadcasted_iota(jnp.int32, (B, 24), 1)
        row = lax.broadcasted_iota(jnp.int32, (B, 24), 0)
        lhs = (jnp.where(lane == 3 * row, xs, 0.0)
               + jnp.where(lane == 3 * row + 1, ys, 0.0)
               + jnp.where(lane == 3 * row + 2, zs, 0.0))
        dot = lax.dot_general(lhs, xyzT24_ref[...], (((1,), (0,)), ((), ())),
                              preferred_element_type=jnp.float32)
        d2 = (ns + n8) - 2.0 * dot
        return jnp.sqrt(jnp.maximum(d2, 0.0)), (xs, ys, zs)

    def store_sel(nn, xs, ys, zs):
        selx_ref[pl.ds(nn, 1)] = xs.reshape(1, B, 1)
        sely_ref[pl.ds(nn, 1)] = ys.reshape(1, B, 1)
        selz_ref[pl.ds(nn, 1)] = zs.reshape(1, B, 1)

    iota_r = lax.broadcasted_iota(jnp.int32, (B, SEED_ROWS, 1), 1)
    iota_c = lax.broadcasted_iota(jnp.int32, (B, SEED_ROWS, N), 2)
    iota_b = lax.broadcasted_iota(jnp.int32, (B, 1), 0)

    def seed_body(rt, carry):
        bv, brow, bcol = carry
        lhs = lhs_seed_ref[rt]
        dot = lax.dot_general(lhs, xyzT24_ref[...], (((1,), (0,)), ((), ())),
                              preferred_element_type=jnp.float32)
        dot3 = dot.reshape(B, SEED_ROWS, N)
        nc = nseed_ref[rt].reshape(B, SEED_ROWS, 1)
        d2 = (nc + n8.reshape(B, 1, N)) - 2.0 * dot3
        dist3 = jnp.sqrt(jnp.maximum(d2, 0.0))
        m2, a2 = _first_argmax(dist3, iota_c, 2)
        m1 = jnp.max(m2, axis=1)
        rsel = jnp.min(jnp.where(m2[..., 0] == m1, iota_r[..., 0],
                                 jnp.int32(N * N)), axis=1, keepdims=True)
        csel = jnp.min(jnp.where(iota_r[..., 0] == rsel, a2[..., 0],
                                 jnp.int32(N * N)), axis=1, keepdims=True)
        upd = m1 > bv
        bv = jnp.where(upd, m1, bv)
        brow = jnp.where(upd, rt * SEED_ROWS + rsel, brow)
        bcol = jnp.where(upd, csel, bcol)
        return bv, brow, bcol

    init = (jnp.full((B, 1), -1.0, jnp.float32),
            jnp.zeros((B, 1), jnp.int32), jnp.zeros((B, 1), jnp.int32))
    _, j0, i0 = lax.fori_loop(0, NUM_SEED_TILES, seed_body, init)

    drow_j0, cj = dist_row(j0)
    store_sel(0, *cj)
    drow_i0, ci = dist_row(i0)
    store_sel(1, *ci)
    dist0 = jnp.minimum(drow_i0, drow_j0)

    def body(nn, dist):
        m = jnp.max(dist, axis=1, keepdims=True)
        i = jnp.min(jnp.where(dist == m, iota_n, jnp.int32(N)), axis=1,
                    keepdims=True)
        drow, coords = dist_row(i)
        store_sel(nn, *coords)
        return jnp.minimum(dist, drow)

    lax.fori_loop(2, NPOINT, body, dist0)


def _bq_dist_kernel(nxyz_ref, ncen_ref, xyzT_ref, n8_ref, out_ref):
    lhs = nxyz_ref[0]
    rhs = xyzT_ref[0]
    dot = lax.dot_general(lhs, rhs, (((1,), (0,)), ((), ())),
                          preferred_element_type=jnp.float32)
    d2 = (ncen_ref[0] + n8_ref[0]) - 2.0 * dot
    out_ref[0] = jnp.sqrt(jnp.maximum(d2, 0.0))


def _run_fps(xyz, n8, interpret=False):
    xT = xyz[..., 0]
    yT = xyz[..., 1]
    zT = xyz[..., 2]
    xyzT24 = xyz.transpose(0, 2, 1).reshape(24, N)
    rows = xyz.reshape(B, NUM_SEED_TILES, SEED_ROWS, 3)
    eye = jnp.eye(B, dtype=jnp.float32)
    lhs_seed = (rows[:, :, :, None, :] * eye[:, None, None, :, None]
                ).transpose(1, 0, 2, 3, 4).reshape(NUM_SEED_TILES,
                                                   B * SEED_ROWS, 24)
    nseed = n8.reshape(B, NUM_SEED_TILES, SEED_ROWS).transpose(1, 0, 2)

    out_shape = [jax.ShapeDtypeStruct((NPOINT, B, 1), jnp.float32)] * 3
    selx, sely, selz = pl.pallas_call(
        _fps_kernel,
        out_shape=out_shape,
        interpret=interpret,
    )(lhs_seed, nseed, xyzT24, xT, yT, zT, n8)
    new_xyz = jnp.concatenate([selx, sely, selz], axis=2).transpose(1, 0, 2)
    return new_xyz


def _run_bq_dist(new_xyz, ncen, xyzT, n8, interpret=False):
    CT = 128
    grid = (B, NPOINT // CT)
    return pl.pallas_call(
        _bq_dist_kernel,
        grid=grid,
        in_specs=[
            pl.BlockSpec((1, CT, 3), lambda b, c: (b, c, 0)),
            pl.BlockSpec((1, CT, 1), lambda b, c: (b, c, 0)),
            pl.BlockSpec((1, 3, N), lambda b, c: (b, 0, 0)),
            pl.BlockSpec((1, 1, N), lambda b, c: (b, 0, 0)),
        ],
        out_specs=pl.BlockSpec((1, CT, N), lambda b, c: (b, c, 0)),
        out_shape=jax.ShapeDtypeStruct((B, NPOINT, N), jnp.float32),
        interpret=interpret,
    )(new_xyz, ncen.reshape(B, NPOINT, 1), xyzT, n8.reshape(B, 1, N))


def kernel(xyz, points, interpret=False):
    n8 = jnp.sum(xyz * xyz, -1)
    new_xyz = _run_fps(xyz, n8, interpret)
    ncen = jnp.sum(new_xyz * new_xyz, -1)
    xyzT = xyz.transpose(0, 2, 1)
    dmat = _run_bq_dist(new_xyz, ncen, xyzT, n8, interpret)

    jidx = jnp.arange(N, dtype=jnp.int32)
    cand = jnp.where(dmat < RADIUS, jidx[None, None, :], jnp.int32(N))
    sorted_idx = jnp.sort(cand, axis=-1)[..., :NSAMPLE]
    first = sorted_idx[..., :1]
    idx = jnp.where(sorted_idx >= N, first, sorted_idx)
    grouped_xyz = jax.vmap(lambda p, i: p[i])(xyz, idx) - new_xyz[:, :, None, :]
    grouped_points = jax.vmap(lambda p, i: p[i])(points, idx)
    new_points = jnp.concatenate([grouped_xyz, grouped_points], axis=-1)
    return (new_xyz, new_points)
